# baseline (device time: 105113 ns/iter reference)
import jax
import jax.numpy as jnp
from jax import lax
from jax.experimental import pallas as pl
from jax.experimental.pallas import tpu as pltpu

N_DEV = 4
SQ = 1024
SQ_SH = 256
HQ_SH = 8
DH = 128
D_MODEL = 1024
KV_LEN = 1152
WIN = 512
KV_STARTS = (0, 128, 384, 640)
SCALE = 0.08838834764831843
NEG = -1e9


def kernel(x, Wq, K_ext, V_ext, Wo):
    my = lax.axis_index("i")
    x2 = x[0]
    k_sl = lax.dynamic_slice(K_ext, (0, 0, my * HQ_SH, 0), (1, KV_LEN, HQ_SH, DH))
    v_sl = lax.dynamic_slice(V_ext, (0, 0, my * HQ_SH, 0), (1, KV_LEN, HQ_SH, DH))
    k_h = jnp.transpose(k_sl[0], (1, 0, 2))
    v_h = jnp.transpose(v_sl[0], (1, 0, 2))

    def body(x_ref, wq_ref, k_ref, v_ref, wo_ref, out_ref,
             xfull, qfull, ctx, part, comm, send_sems, recv_sems):
        my_pos = lax.axis_index("i")
        left = (my_pos - 1) % N_DEV
        right = (my_pos + 1) % N_DEV

        barrier_sem = pltpu.get_barrier_semaphore()
        for nbr in [left, right]:
            pl.semaphore_signal(
                barrier_sem, inc=1,
                device_id=(nbr,), device_id_type=pl.DeviceIdType.MESH,
            )
        pl.semaphore_wait(barrier_sem, 2)

        xfull[pl.ds(my_pos * SQ_SH, SQ_SH), :] = x_ref[:, :]
        comm[0, :, :] = x_ref[:, :]
        for h in range(N_DEV - 1):
            s_slot, r_slot = h % 2, (h + 1) % 2
            rdma = pltpu.make_async_remote_copy(
                src_ref=comm.at[s_slot],
                dst_ref=comm.at[r_slot],
                send_sem=send_sems.at[s_slot],
                recv_sem=recv_sems.at[r_slot],
                device_id=(right,),
                device_id_type=pl.DeviceIdType.MESH,
            )
            rdma.start()
            rdma.wait()
            origin = (my_pos - h - 1) % N_DEV
            xfull[pl.ds(origin * SQ_SH, SQ_SH), :] = comm[r_slot, :, :]

        qfull[:, :] = jnp.dot(
            xfull[:, :], wq_ref[:, :], preferred_element_type=jnp.float32
        )

        for b in range(N_DEV):
            s_b = KV_STARTS[b]
            rows = lax.broadcasted_iota(jnp.int32, (SQ_SH, WIN), 0) + b * SQ_SH
            cols = lax.broadcasted_iota(jnp.int32, (SQ_SH, WIN), 1) + s_b
            mask = jnp.abs(rows - cols) <= 128
            for hh in range(HQ_SH):
                q_bh = qfull[b * SQ_SH:(b + 1) * SQ_SH, hh * DH:(hh + 1) * DH]
                k_bh = k_ref[hh, s_b:s_b + WIN, :]
                v_bh = v_ref[hh, s_b:s_b + WIN, :]
                s = lax.dot_general(
                    q_bh, k_bh, (((1,), (1,)), ((), ())),
                    preferred_element_type=jnp.float32,
                ) * SCALE
                s = jnp.where(mask, s, NEG)
                m = jnp.max(s, axis=1, keepdims=True)
                w = jnp.exp(s - m)
                w = w / jnp.sum(w, axis=1, keepdims=True)
                ctx[b * SQ_SH:(b + 1) * SQ_SH, hh * DH:(hh + 1) * DH] = jnp.dot(
                    w, v_bh, preferred_element_type=jnp.float32
                )

        part[:, :] = jnp.dot(
            ctx[:, :], wo_ref[:, :], preferred_element_type=jnp.float32
        )

        comm[0, :, :] = part[pl.ds(((my_pos + 3) % N_DEV) * SQ_SH, SQ_SH), :]
        for st in range(N_DEV - 1):
            s_slot, r_slot = st % 2, (st + 1) % 2
            rdma = pltpu.make_async_remote_copy(
                src_ref=comm.at[s_slot],
                dst_ref=comm.at[r_slot],
                send_sem=send_sems.at[s_slot],
                recv_sem=recv_sems.at[r_slot],
                device_id=(right,),
                device_id_type=pl.DeviceIdType.MESH,
            )
            rdma.start()
            rdma.wait()
            c_recv = (my_pos + 2 - st) % N_DEV
            if st < N_DEV - 2:
                comm[r_slot, :, :] = (
                    comm[r_slot, :, :] + part[pl.ds(c_recv * SQ_SH, SQ_SH), :]
                )
            else:
                out_ref[0, :, :] = (
                    comm[r_slot, :, :] + part[pl.ds(my_pos * SQ_SH, SQ_SH), :]
                )

    out = pl.pallas_call(
        body,
        out_shape=jax.ShapeDtypeStruct((1, SQ_SH, D_MODEL), jnp.float32),
        in_specs=[pl.BlockSpec(memory_space=pltpu.VMEM)] * 5,
        out_specs=pl.BlockSpec(memory_space=pltpu.VMEM),
        scratch_shapes=[
            pltpu.VMEM((SQ, D_MODEL), jnp.float32),
            pltpu.VMEM((SQ, HQ_SH * DH), jnp.float32),
            pltpu.VMEM((SQ, HQ_SH * DH), jnp.float32),
            pltpu.VMEM((SQ, D_MODEL), jnp.float32),
            pltpu.VMEM((2, SQ_SH, D_MODEL), jnp.float32),
            pltpu.SemaphoreType.DMA((2,)),
            pltpu.SemaphoreType.DMA((2,)),
        ],
        compiler_params=pltpu.CompilerParams(collective_id=0),
    )(x2, Wq, k_h, v_h, Wo)
    return out


# device time: 97651 ns/iter; 1.0764x vs baseline; 1.0764x over previous
import jax
import jax.numpy as jnp
from jax import lax
from jax.experimental import pallas as pl
from jax.experimental.pallas import tpu as pltpu

N_DEV = 4
SQ = 1024
SQ_SH = 256
HQ_SH = 8
DH = 128
D_MODEL = 1024
KV_LEN = 1152
WIN = 512
KV_STARTS = (0, 128, 384, 640)
SCALE = 0.08838834764831843
NEG = -1e9


def kernel(x, Wq, K_ext, V_ext, Wo):
    my = lax.axis_index("i")
    x2 = x[0]
    k_sl = lax.dynamic_slice(K_ext, (0, 0, my * HQ_SH, 0), (1, KV_LEN, HQ_SH, DH))
    v_sl = lax.dynamic_slice(V_ext, (0, 0, my * HQ_SH, 0), (1, KV_LEN, HQ_SH, DH))
    k_h = jnp.transpose(k_sl[0], (1, 0, 2))
    v_h = jnp.transpose(v_sl[0], (1, 0, 2))
    k_win = jnp.stack([k_h[:, s:s + WIN, :] for s in KV_STARTS], axis=0)
    v_win = jnp.stack([v_h[:, s:s + WIN, :] for s in KV_STARTS], axis=0)

    def body(x_ref, wq_ref, kw_ref, vw_ref, wo_ref, out_ref,
             ag_comm, rs_comm, part,
             ag_ssem, ag_rsem, rs_ssem, rs_rsem):
        my_pos = lax.axis_index("i")
        left = (my_pos - 1) % N_DEV
        right = (my_pos + 1) % N_DEV

        barrier_sem = pltpu.get_barrier_semaphore()
        for nbr in [left, right]:
            pl.semaphore_signal(
                barrier_sem, inc=1,
                device_id=(nbr,), device_id_type=pl.DeviceIdType.MESH,
            )
        pl.semaphore_wait(barrier_sem, 2)

        def mk_ag(h):
            return pltpu.make_async_remote_copy(
                src_ref=(x_ref if h == 0 else ag_comm.at[h - 1]),
                dst_ref=ag_comm.at[h],
                send_sem=ag_ssem.at[h],
                recv_sem=ag_rsem.at[h],
                device_id=(right,),
                device_id_type=pl.DeviceIdType.MESH,
            )

        def mk_rs(s):
            return pltpu.make_async_remote_copy(
                src_ref=rs_comm.at[s],
                dst_ref=rs_comm.at[s + 1],
                send_sem=rs_ssem.at[s],
                recv_sem=rs_rsem.at[s],
                device_id=(right,),
                device_id_type=pl.DeviceIdType.MESH,
            )

        ags = [mk_ag(h) for h in range(N_DEV - 1)]
        rss = [mk_rs(s) for s in range(N_DEV - 1)]

        def compute_chunk(xc, origin, dst):
            xq = jnp.dot(xc, wq_ref[:, :], preferred_element_type=jnp.float32)
            delta = jnp.where(origin == 0, 0, 128)
            rows = lax.broadcasted_iota(jnp.int32, (SQ_SH, WIN), 0) + delta
            cols = lax.broadcasted_iota(jnp.int32, (SQ_SH, WIN), 1)
            mask = jnp.abs(rows - cols) <= 128
            ctx_parts = []
            for hh in range(HQ_SH):
                q_bh = xq[:, hh * DH:(hh + 1) * DH]
                k_bh = kw_ref[origin, hh, :, :]
                v_bh = vw_ref[origin, hh, :, :]
                s = lax.dot_general(
                    q_bh, k_bh, (((1,), (1,)), ((), ())),
                    preferred_element_type=jnp.float32,
                ) * SCALE
                s = jnp.where(mask, s, NEG)
                m = jnp.max(s, axis=1, keepdims=True)
                w = jnp.exp(s - m)
                w = w / jnp.sum(w, axis=1, keepdims=True)
                ctx_parts.append(
                    jnp.dot(w, v_bh, preferred_element_type=jnp.float32)
                )
            ctx = jnp.concatenate(ctx_parts, axis=1)
            dst[:, :] = jnp.dot(
                ctx, wo_ref[:, :], preferred_element_type=jnp.float32
            )

        ags[0].start()
        compute_chunk(x_ref[:, :], my_pos, part.at[0])

        ags[0].wait_recv()
        ags[1].start()
        compute_chunk(ag_comm[0, :, :], (my_pos + 3) % N_DEV, rs_comm.at[0])
        rss[0].start()

        ags[1].wait_recv()
        ags[2].start()
        compute_chunk(ag_comm[1, :, :], (my_pos + 2) % N_DEV, part.at[1])
        rss[0].wait_recv()
        rs_comm[1, :, :] = rs_comm[1, :, :] + part[1, :, :]
        rss[1].start()

        ags[2].wait_recv()
        compute_chunk(ag_comm[2, :, :], (my_pos + 1) % N_DEV, part.at[2])
        rss[1].wait_recv()
        rs_comm[2, :, :] = rs_comm[2, :, :] + part[2, :, :]
        rss[2].start()

        rss[2].wait_recv()
        out_ref[0, :, :] = rs_comm[3, :, :] + part[0, :, :]

        for d in ags:
            d.wait_send()
        for d in rss:
            d.wait_send()

    out = pl.pallas_call(
        body,
        out_shape=jax.ShapeDtypeStruct((1, SQ_SH, D_MODEL), jnp.float32),
        in_specs=[pl.BlockSpec(memory_space=pltpu.VMEM)] * 5,
        out_specs=pl.BlockSpec(memory_space=pltpu.VMEM),
        scratch_shapes=[
            pltpu.VMEM((N_DEV - 1, SQ_SH, D_MODEL), jnp.float32),
            pltpu.VMEM((N_DEV, SQ_SH, D_MODEL), jnp.float32),
            pltpu.VMEM((3, SQ_SH, D_MODEL), jnp.float32),
            pltpu.SemaphoreType.DMA((N_DEV - 1,)),
            pltpu.SemaphoreType.DMA((N_DEV - 1,)),
            pltpu.SemaphoreType.DMA((N_DEV - 1,)),
            pltpu.SemaphoreType.DMA((N_DEV - 1,)),
        ],
        compiler_params=pltpu.CompilerParams(collective_id=0),
    )(x2, Wq, k_win, v_win, Wo)
    return out


# device time: 57639 ns/iter; 1.8236x vs baseline; 1.6942x over previous
import jax
import jax.numpy as jnp
from jax import lax
from jax.experimental import pallas as pl
from jax.experimental.pallas import tpu as pltpu

N_DEV = 4
SQ = 1024
SQ_SH = 256
HQ_SH = 8
DH = 128
D_MODEL = 1024
KV_LEN = 1152
WIN = 512
KV_STARTS = (0, 128, 384, 640)
SCALE = 0.08838834764831843
NEG = -1e9

BF = jnp.bfloat16
F32 = jnp.float32


def kernel(x, Wq, K_ext, V_ext, Wo):
    my = lax.axis_index("i")
    x2 = x[0].astype(BF)
    wq = Wq.astype(BF)
    wo = Wo.astype(BF)
    k_sl = lax.dynamic_slice(K_ext, (0, 0, my * HQ_SH, 0), (1, KV_LEN, HQ_SH, DH))
    v_sl = lax.dynamic_slice(V_ext, (0, 0, my * HQ_SH, 0), (1, KV_LEN, HQ_SH, DH))
    k_h = jnp.transpose(k_sl[0], (1, 0, 2)).astype(BF)
    v_h = jnp.transpose(v_sl[0], (1, 0, 2)).astype(BF)
    k_win = jnp.stack([k_h[:, s:s + WIN, :] for s in KV_STARTS], axis=0)
    v_win = jnp.stack([v_h[:, s:s + WIN, :] for s in KV_STARTS], axis=0)

    def body(x_ref, wq_ref, kw_ref, vw_ref, wo_ref, out_ref,
             ag_comm, rs_comm, part,
             ag_ssem, ag_rsem, rs_ssem, rs_rsem):
        my_pos = lax.axis_index("i")
        left = (my_pos - 1) % N_DEV
        right = (my_pos + 1) % N_DEV

        barrier_sem = pltpu.get_barrier_semaphore()
        for nbr in [left, right]:
            pl.semaphore_signal(
                barrier_sem, inc=1,
                device_id=(nbr,), device_id_type=pl.DeviceIdType.MESH,
            )
        pl.semaphore_wait(barrier_sem, 2)

        def mk_ag(h):
            return pltpu.make_async_remote_copy(
                src_ref=(x_ref if h == 0 else ag_comm.at[h - 1]),
                dst_ref=ag_comm.at[h],
                send_sem=ag_ssem.at[h],
                recv_sem=ag_rsem.at[h],
                device_id=(right,),
                device_id_type=pl.DeviceIdType.MESH,
            )

        def mk_rs(s):
            return pltpu.make_async_remote_copy(
                src_ref=rs_comm.at[s],
                dst_ref=rs_comm.at[s + 1],
                send_sem=rs_ssem.at[s],
                recv_sem=rs_rsem.at[s],
                device_id=(right,),
                device_id_type=pl.DeviceIdType.MESH,
            )

        ags = [mk_ag(h) for h in range(N_DEV - 1)]
        rss = [mk_rs(s) for s in range(N_DEV - 1)]

        def compute_chunk(xc, origin, dst):
            xq = jnp.dot(
                xc, wq_ref[:, :], preferred_element_type=F32
            ).astype(BF)
            delta = jnp.where(origin == 0, 0, 128)
            rows = lax.broadcasted_iota(jnp.int32, (SQ_SH, WIN), 0) + delta
            cols = lax.broadcasted_iota(jnp.int32, (SQ_SH, WIN), 1)
            mask = jnp.abs(rows - cols) <= 128
            ctx_parts = []
            for hh in range(HQ_SH):
                q_bh = xq[:, hh * DH:(hh + 1) * DH]
                k_bh = kw_ref[origin, hh, :, :]
                v_bh = vw_ref[origin, hh, :, :]
                s = lax.dot_general(
                    q_bh, k_bh, (((1,), (1,)), ((), ())),
                    preferred_element_type=F32,
                ) * SCALE
                s = jnp.where(mask, s, NEG)
                m = jnp.max(s, axis=1, keepdims=True)
                w = jnp.exp(s - m)
                w = (w / jnp.sum(w, axis=1, keepdims=True)).astype(BF)
                ctx_parts.append(
                    jnp.dot(w, v_bh, preferred_element_type=F32).astype(BF)
                )
            ctx = jnp.concatenate(ctx_parts, axis=1)
            dst[:, :] = jnp.dot(
                ctx, wo_ref[:, :], preferred_element_type=F32
            ).astype(BF)

        ags[0].start()
        compute_chunk(x_ref[:, :], my_pos, part.at[0])

        ags[0].wait_recv()
        ags[1].start()
        compute_chunk(ag_comm[0, :, :], (my_pos + 3) % N_DEV, rs_comm.at[0])
        rss[0].start()

        ags[1].wait_recv()
        ags[2].start()
        compute_chunk(ag_comm[1, :, :], (my_pos + 2) % N_DEV, part.at[1])
        rss[0].wait_recv()
        rs_comm[1, :, :] = (
            rs_comm[1, :, :].astype(F32) + part[1, :, :].astype(F32)
        ).astype(BF)
        rss[1].start()

        ags[2].wait_recv()
        compute_chunk(ag_comm[2, :, :], (my_pos + 1) % N_DEV, part.at[2])
        rss[1].wait_recv()
        rs_comm[2, :, :] = (
            rs_comm[2, :, :].astype(F32) + part[2, :, :].astype(F32)
        ).astype(BF)
        rss[2].start()

        rss[2].wait_recv()
        out_ref[0, :, :] = (
            rs_comm[3, :, :].astype(F32) + part[0, :, :].astype(F32)
        )

        for d in ags:
            d.wait_send()
        for d in rss:
            d.wait_send()

    out = pl.pallas_call(
        body,
        out_shape=jax.ShapeDtypeStruct((1, SQ_SH, D_MODEL), F32),
        in_specs=[pl.BlockSpec(memory_space=pltpu.VMEM)] * 5,
        out_specs=pl.BlockSpec(memory_space=pltpu.VMEM),
        scratch_shapes=[
            pltpu.VMEM((N_DEV - 1, SQ_SH, D_MODEL), BF),
            pltpu.VMEM((N_DEV, SQ_SH, D_MODEL), BF),
            pltpu.VMEM((3, SQ_SH, D_MODEL), BF),
            pltpu.SemaphoreType.DMA((N_DEV - 1,)),
            pltpu.SemaphoreType.DMA((N_DEV - 1,)),
            pltpu.SemaphoreType.DMA((N_DEV - 1,)),
            pltpu.SemaphoreType.DMA((N_DEV - 1,)),
        ],
        compiler_params=pltpu.CompilerParams(collective_id=0),
    )(x2, wq, k_win, v_win, wo)
    return out


# device time: 48015 ns/iter; 2.1892x vs baseline; 1.2004x over previous
import jax
import jax.numpy as jnp
from jax import lax
from jax.experimental import pallas as pl
from jax.experimental.pallas import tpu as pltpu

N_DEV = 4
SQ = 1024
SQ_SH = 256
HQ_SH = 8
DH = 128
D_MODEL = 1024
KV_LEN = 1152
WIN = 512
SCALE = 0.08838834764831843
NEG = -1e9

BF = jnp.bfloat16
F32 = jnp.float32


def kernel(x, Wq, K_ext, V_ext, Wo):
    x2 = x[0].astype(BF)

    def body(x_ref, wq_hbm, k_hbm, v_hbm, wo_hbm, out_ref,
             kbuf, vbuf, kb16, vb16, wqf, wqb, wof, wob,
             ag_cw, ag_ccw, rs_cw, rs_ccw, part,
             ksems, vsems, wsems,
             ag_ssem, ag_rsem, rs_ssem, rs_rsem):
        my_pos = lax.axis_index("i")
        left = (my_pos - 1) % N_DEV
        right = (my_pos + 1) % N_DEV

        kv_copies = []
        for hh in range(HQ_SH):
            for (hbm, buf, sems) in ((k_hbm, kbuf, ksems), (v_hbm, vbuf, vsems)):
                c = pltpu.make_async_copy(
                    hbm.at[0, pl.ds(0, KV_LEN), my_pos * HQ_SH + hh, :],
                    buf.at[hh],
                    sems.at[hh],
                )
                c.start()
                kv_copies.append(c)
        wq_copy = pltpu.make_async_copy(wq_hbm, wqf, wsems.at[0])
        wo_copy = pltpu.make_async_copy(wo_hbm, wof, wsems.at[1])
        wq_copy.start()
        wo_copy.start()

        barrier_sem = pltpu.get_barrier_semaphore()
        for nbr in [left, right]:
            pl.semaphore_signal(
                barrier_sem, inc=1,
                device_id=(nbr,), device_id_type=pl.DeviceIdType.MESH,
            )
        pl.semaphore_wait(barrier_sem, 2)

        def rdma(src, dst, ssem, rsem, dev):
            return pltpu.make_async_remote_copy(
                src_ref=src, dst_ref=dst, send_sem=ssem, recv_sem=rsem,
                device_id=(dev,), device_id_type=pl.DeviceIdType.MESH,
            )

        d_ag_cw0 = rdma(x_ref, ag_cw.at[0], ag_ssem.at[0], ag_rsem.at[0], right)
        d_ag_cw1 = rdma(ag_cw.at[0], ag_cw.at[1], ag_ssem.at[1], ag_rsem.at[1], right)
        d_ag_ccw = rdma(x_ref, ag_ccw.at[0], ag_ssem.at[2], ag_rsem.at[2], left)
        d_rs_cw0 = rdma(rs_cw.at[0], rs_cw.at[1], rs_ssem.at[0], rs_rsem.at[0], right)
        d_rs_cw1 = rdma(rs_cw.at[1], rs_cw.at[2], rs_ssem.at[1], rs_rsem.at[1], right)
        d_rs_ccw = rdma(rs_ccw.at[0], rs_ccw.at[1], rs_ssem.at[2], rs_rsem.at[2], left)

        d_ag_cw0.start()
        d_ag_ccw.start()
        wq_copy.wait()
        wqb[:, :] = wqf[:, :].astype(BF)
        wo_copy.wait()
        wob[:, :] = wof[:, :].astype(BF)
        for c in kv_copies:
            c.wait()
        kb16[:, :, :] = kbuf[:, :, :].astype(BF)
        vb16[:, :, :] = vbuf[:, :, :].astype(BF)

        def compute_chunk(xc, origin, dst):
            xq = jnp.dot(xc, wqb[:, :], preferred_element_type=F32).astype(BF)
            s_b = pl.multiple_of(
                jnp.clip(origin * SQ_SH - 128, 0, KV_LEN - WIN), 128
            )
            delta = jnp.where(origin == 0, 0, 128)
            rows = lax.broadcasted_iota(jnp.int32, (SQ_SH, WIN), 0) + delta
            cols = lax.broadcasted_iota(jnp.int32, (SQ_SH, WIN), 1)
            mask = jnp.abs(rows - cols) <= 128
            ctx_parts = []
            for hh in range(HQ_SH):
                q_bh = xq[:, hh * DH:(hh + 1) * DH]
                k_bh = kb16[hh, pl.ds(s_b, WIN), :]
                v_bh = vb16[hh, pl.ds(s_b, WIN), :]
                s = lax.dot_general(
                    q_bh, k_bh, (((1,), (1,)), ((), ())),
                    preferred_element_type=F32,
                ) * SCALE
                s = jnp.where(mask, s, NEG)
                m = jnp.max(s, axis=1, keepdims=True)
                w = jnp.exp(s - m)
                w = (w / jnp.sum(w, axis=1, keepdims=True)).astype(BF)
                ctx_parts.append(
                    jnp.dot(w, v_bh, preferred_element_type=F32).astype(BF)
                )
            ctx = jnp.concatenate(ctx_parts, axis=1)
            dst[:, :] = jnp.dot(
                ctx, wob[:, :], preferred_element_type=F32
            ).astype(BF)

        compute_chunk(x_ref[:, :], my_pos, part.at[0])

        d_ag_cw0.wait_recv()
        d_ag_cw1.start()
        compute_chunk(ag_cw[0, :, :], (my_pos + 3) % N_DEV, rs_ccw.at[0])
        d_rs_ccw.start()

        d_ag_cw1.wait_recv()
        compute_chunk(ag_cw[1, :, :], (my_pos + 2) % N_DEV, rs_cw.at[0])
        d_rs_cw0.start()

        d_ag_ccw.wait_recv()
        compute_chunk(ag_ccw[0, :, :], (my_pos + 1) % N_DEV, part.at[1])

        d_rs_cw0.wait_recv()
        rs_cw[1, :, :] = (
            rs_cw[1, :, :].astype(F32) + part[1, :, :].astype(F32)
        ).astype(BF)
        d_rs_cw1.start()

        d_rs_cw1.wait_recv()
        d_rs_ccw.wait_recv()
        out_ref[0, :, :] = (
            part[0, :, :].astype(F32)
            + rs_cw[2, :, :].astype(F32)
            + rs_ccw[1, :, :].astype(F32)
        )

        for d in [d_ag_cw0, d_ag_cw1, d_ag_ccw, d_rs_cw0, d_rs_cw1, d_rs_ccw]:
            d.wait_send()

    out = pl.pallas_call(
        body,
        out_shape=jax.ShapeDtypeStruct((1, SQ_SH, D_MODEL), F32),
        in_specs=[
            pl.BlockSpec(memory_space=pltpu.VMEM),
            pl.BlockSpec(memory_space=pl.ANY),
            pl.BlockSpec(memory_space=pl.ANY),
            pl.BlockSpec(memory_space=pl.ANY),
            pl.BlockSpec(memory_space=pl.ANY),
        ],
        out_specs=pl.BlockSpec(memory_space=pltpu.VMEM),
        scratch_shapes=[
            pltpu.VMEM((HQ_SH, KV_LEN, DH), F32),
            pltpu.VMEM((HQ_SH, KV_LEN, DH), F32),
            pltpu.VMEM((HQ_SH, KV_LEN, DH), BF),
            pltpu.VMEM((HQ_SH, KV_LEN, DH), BF),
            pltpu.VMEM((D_MODEL, D_MODEL), F32),
            pltpu.VMEM((D_MODEL, D_MODEL), BF),
            pltpu.VMEM((D_MODEL, D_MODEL), F32),
            pltpu.VMEM((D_MODEL, D_MODEL), BF),
            pltpu.VMEM((2, SQ_SH, D_MODEL), BF),
            pltpu.VMEM((1, SQ_SH, D_MODEL), BF),
            pltpu.VMEM((3, SQ_SH, D_MODEL), BF),
            pltpu.VMEM((2, SQ_SH, D_MODEL), BF),
            pltpu.VMEM((2, SQ_SH, D_MODEL), BF),
            pltpu.SemaphoreType.DMA((HQ_SH,)),
            pltpu.SemaphoreType.DMA((HQ_SH,)),
            pltpu.SemaphoreType.DMA((2,)),
            pltpu.SemaphoreType.DMA((3,)),
            pltpu.SemaphoreType.DMA((3,)),
            pltpu.SemaphoreType.DMA((3,)),
            pltpu.SemaphoreType.DMA((3,)),
        ],
        compiler_params=pltpu.CompilerParams(
            collective_id=0, vmem_limit_bytes=100 * 1024 * 1024
        ),
    )(x2, Wq, K_ext, V_ext, Wo)
    return out


# device time: 44299 ns/iter; 2.3728x vs baseline; 1.0839x over previous
import jax
import jax.numpy as jnp
from jax import lax
from jax.experimental import pallas as pl
from jax.experimental.pallas import tpu as pltpu

N_DEV = 4
SQ = 1024
SQ_SH = 256
HQ_SH = 8
DH = 128
D_MODEL = 1024
KV_LEN = 1152
WIN = 512
SCALE = 0.08838834764831843
NEG = -1e9

BF = jnp.bfloat16
F32 = jnp.float32


def kernel(x, Wq, K_ext, V_ext, Wo):
    x2 = x[0].astype(BF)

    def body(x_ref, wq_hbm, k_hbm, v_hbm, wo_hbm, out_ref,
             kbuf, vbuf, kb16, vb16, wqf, wqb, wof, wob,
             agb, rsb, part,
             ksems, vsems, wsems,
             ag_ssem, ag_rsem, rs_ssem, rs_rsem):
        my_pos = lax.axis_index("i")
        left = (my_pos - 1) % N_DEV
        right = (my_pos + 1) % N_DEV

        kv_copies = []
        for hh in range(HQ_SH):
            for (hbm, buf, sems) in ((k_hbm, kbuf, ksems), (v_hbm, vbuf, vsems)):
                c = pltpu.make_async_copy(
                    hbm.at[0, pl.ds(0, KV_LEN), my_pos * HQ_SH + hh, :],
                    buf.at[hh],
                    sems.at[hh],
                )
                c.start()
                kv_copies.append(c)
        wq_copy = pltpu.make_async_copy(wq_hbm, wqf, wsems.at[0])
        wo_copy = pltpu.make_async_copy(wo_hbm, wof, wsems.at[1])
        wq_copy.start()
        wo_copy.start()

        barrier_sem = pltpu.get_barrier_semaphore()
        for nbr in [left, right, (my_pos + 2) % N_DEV]:
            pl.semaphore_signal(
                barrier_sem, inc=1,
                device_id=(nbr,), device_id_type=pl.DeviceIdType.MESH,
            )
        pl.semaphore_wait(barrier_sem, 3)

        def rdma(src, dst, ssem, rsem, dev):
            return pltpu.make_async_remote_copy(
                src_ref=src, dst_ref=dst, send_sem=ssem, recv_sem=rsem,
                device_id=(dev,), device_id_type=pl.DeviceIdType.MESH,
            )

        diag = (my_pos + 2) % N_DEV

        d_ag_r = rdma(x_ref, agb.at[0], ag_ssem.at[0], ag_rsem.at[0], right)
        d_ag_l = rdma(x_ref, agb.at[1], ag_ssem.at[1], ag_rsem.at[1], left)
        d_ag_d = rdma(x_ref, agb.at[2], ag_ssem.at[2], ag_rsem.at[2], diag)
        d_rs_r = rdma(part.at[3], rsb.at[0], rs_ssem.at[0], rs_rsem.at[0], right)
        d_rs_l = rdma(part.at[1], rsb.at[1], rs_ssem.at[1], rs_rsem.at[1], left)
        d_rs_d = rdma(part.at[2], rsb.at[2], rs_ssem.at[2], rs_rsem.at[2], diag)

        d_ag_r.start()
        d_ag_l.start()
        d_ag_d.start()
        wq_copy.wait()
        wqb[:, :] = wqf[:, :].astype(BF)
        wo_copy.wait()
        wob[:, :] = wof[:, :].astype(BF)
        for c in kv_copies:
            c.wait()
        kb16[:, :, :] = kbuf[:, :, :].astype(BF)
        vb16[:, :, :] = vbuf[:, :, :].astype(BF)

        def compute_chunk(xc, origin, dst):
            xq = jnp.dot(xc, wqb[:, :], preferred_element_type=F32).astype(BF)
            s_b = pl.multiple_of(
                jnp.clip(origin * SQ_SH - 128, 0, KV_LEN - WIN), 128
            )
            delta = jnp.where(origin == 0, 0, 128)
            rows = lax.broadcasted_iota(jnp.int32, (SQ_SH, WIN), 0) + delta
            cols = lax.broadcasted_iota(jnp.int32, (SQ_SH, WIN), 1)
            mask = jnp.abs(rows - cols) <= 128
            ctx_parts = []
            for hh in range(HQ_SH):
                q_bh = xq[:, hh * DH:(hh + 1) * DH]
                k_bh = kb16[hh, pl.ds(s_b, WIN), :]
                v_bh = vb16[hh, pl.ds(s_b, WIN), :]
                s = lax.dot_general(
                    q_bh, k_bh, (((1,), (1,)), ((), ())),
                    preferred_element_type=F32,
                ) * SCALE
                s = jnp.where(mask, s, NEG)
                m = jnp.max(s, axis=1, keepdims=True)
                w = jnp.exp(s - m)
                w = (w / jnp.sum(w, axis=1, keepdims=True)).astype(BF)
                ctx_parts.append(
                    jnp.dot(w, v_bh, preferred_element_type=F32).astype(BF)
                )
            ctx = jnp.concatenate(ctx_parts, axis=1)
            dst[:, :] = jnp.dot(
                ctx, wob[:, :], preferred_element_type=F32
            ).astype(BF)

        compute_chunk(x_ref[:, :], my_pos, part.at[0])

        d_ag_r.wait_recv()
        compute_chunk(agb[0, :, :], (my_pos + 3) % N_DEV, part.at[1])
        d_rs_l.start()

        d_ag_d.wait_recv()
        compute_chunk(agb[2, :, :], (my_pos + 2) % N_DEV, part.at[2])
        d_rs_d.start()

        d_ag_l.wait_recv()
        compute_chunk(agb[1, :, :], (my_pos + 1) % N_DEV, part.at[3])
        d_rs_r.start()

        d_rs_r.wait_recv()
        d_rs_l.wait_recv()
        d_rs_d.wait_recv()
        out_ref[0, :, :] = (
            (part[0, :, :].astype(F32) + rsb[0, :, :].astype(F32))
            + (rsb[1, :, :].astype(F32) + rsb[2, :, :].astype(F32))
        )

        for d in [d_ag_r, d_ag_l, d_ag_d, d_rs_r, d_rs_l, d_rs_d]:
            d.wait_send()

    out = pl.pallas_call(
        body,
        out_shape=jax.ShapeDtypeStruct((1, SQ_SH, D_MODEL), F32),
        in_specs=[
            pl.BlockSpec(memory_space=pltpu.VMEM),
            pl.BlockSpec(memory_space=pl.ANY),
            pl.BlockSpec(memory_space=pl.ANY),
            pl.BlockSpec(memory_space=pl.ANY),
            pl.BlockSpec(memory_space=pl.ANY),
        ],
        out_specs=pl.BlockSpec(memory_space=pltpu.VMEM),
        scratch_shapes=[
            pltpu.VMEM((HQ_SH, KV_LEN, DH), F32),
            pltpu.VMEM((HQ_SH, KV_LEN, DH), F32),
            pltpu.VMEM((HQ_SH, KV_LEN, DH), BF),
            pltpu.VMEM((HQ_SH, KV_LEN, DH), BF),
            pltpu.VMEM((D_MODEL, D_MODEL), F32),
            pltpu.VMEM((D_MODEL, D_MODEL), BF),
            pltpu.VMEM((D_MODEL, D_MODEL), F32),
            pltpu.VMEM((D_MODEL, D_MODEL), BF),
            pltpu.VMEM((3, SQ_SH, D_MODEL), BF),
            pltpu.VMEM((3, SQ_SH, D_MODEL), BF),
            pltpu.VMEM((4, SQ_SH, D_MODEL), BF),
            pltpu.SemaphoreType.DMA((HQ_SH,)),
            pltpu.SemaphoreType.DMA((HQ_SH,)),
            pltpu.SemaphoreType.DMA((2,)),
            pltpu.SemaphoreType.DMA((3,)),
            pltpu.SemaphoreType.DMA((3,)),
            pltpu.SemaphoreType.DMA((3,)),
            pltpu.SemaphoreType.DMA((3,)),
        ],
        compiler_params=pltpu.CompilerParams(
            collective_id=0, vmem_limit_bytes=100 * 1024 * 1024
        ),
    )(x2, Wq, K_ext, V_ext, Wo)
    return out


# device time: 41756 ns/iter; 2.5173x vs baseline; 1.0609x over previous
import jax
import jax.numpy as jnp
from jax import lax
from jax.experimental import pallas as pl
from jax.experimental.pallas import tpu as pltpu

N_DEV = 4
SQ = 1024
SQ_SH = 256
HQ_SH = 8
DH = 128
D_MODEL = 1024
KV_LEN = 1152
WIN = 512
SCALE = 0.08838834764831843
NEG = -1e9

BF = jnp.bfloat16
F32 = jnp.float32


def kernel(x, Wq, K_ext, V_ext, Wo):
    x2 = x[0].astype(BF)

    def body(x_ref, wq_hbm, k_hbm, v_hbm, wo_hbm, out_ref,
             kbuf, vbuf, kb16, vb16, wqf, wqb, wof, wob,
             agb, rsb, part,
             ksems, vsems, wsems,
             ag_ssem, ag_rsem, rs_ssem, rs_rsem):
        my_pos = lax.axis_index("i")
        left = (my_pos - 1) % N_DEV
        right = (my_pos + 1) % N_DEV

        kv_copies = []
        for hh in range(HQ_SH):
            for (hbm, buf, sems) in ((k_hbm, kbuf, ksems), (v_hbm, vbuf, vsems)):
                c = pltpu.make_async_copy(
                    hbm.at[0, pl.ds(0, KV_LEN), my_pos * HQ_SH + hh, :],
                    buf.at[hh],
                    sems.at[hh],
                )
                c.start()
                kv_copies.append(c)
        wq_copy = pltpu.make_async_copy(wq_hbm, wqf, wsems.at[0])
        wo_copy = pltpu.make_async_copy(wo_hbm, wof, wsems.at[1])
        wq_copy.start()
        wo_copy.start()

        barrier_sem = pltpu.get_barrier_semaphore()
        for nbr in [left, right, (my_pos + 2) % N_DEV]:
            pl.semaphore_signal(
                barrier_sem, inc=1,
                device_id=(nbr,), device_id_type=pl.DeviceIdType.MESH,
            )
        pl.semaphore_wait(barrier_sem, 3)

        def rdma(src, dst, ssem, rsem, dev):
            return pltpu.make_async_remote_copy(
                src_ref=src, dst_ref=dst, send_sem=ssem, recv_sem=rsem,
                device_id=(dev,), device_id_type=pl.DeviceIdType.MESH,
            )

        diag = (my_pos + 2) % N_DEV

        d_ag_r = rdma(x_ref, agb.at[0], ag_ssem.at[0], ag_rsem.at[0], right)
        d_ag_l = rdma(x_ref, agb.at[1], ag_ssem.at[1], ag_rsem.at[1], left)
        d_ag_d = rdma(x_ref, agb.at[2], ag_ssem.at[2], ag_rsem.at[2], diag)
        d_rs_r = rdma(part.at[3], rsb.at[0], rs_ssem.at[0], rs_rsem.at[0], right)
        d_rs_l = rdma(part.at[1], rsb.at[1], rs_ssem.at[1], rs_rsem.at[1], left)
        d_rs_d = rdma(part.at[2], rsb.at[2], rs_ssem.at[2], rs_rsem.at[2], diag)

        d_ag_r.start()
        d_ag_l.start()
        d_ag_d.start()
        wq_copy.wait()
        wqb[:, :] = wqf[:, :].astype(BF)
        wo_copy.wait()
        wob[:, :] = wof[:, :].astype(BF)
        for c in kv_copies:
            c.wait()
        kb16[:, :, :] = kbuf[:, :, :].astype(BF)
        vb16[:, :, :] = vbuf[:, :, :].astype(BF)

        def compute_chunk(xc, origin, dst):
            xq = jnp.dot(xc, wqb[:, :], preferred_element_type=F32).astype(BF)
            s_b = pl.multiple_of(
                jnp.clip(origin * SQ_SH - 128, 0, KV_LEN - WIN), 128
            )
            delta = jnp.where(origin == 0, 0, 128)
            rows = lax.broadcasted_iota(jnp.int32, (SQ_SH, WIN), 0) + delta
            cols = lax.broadcasted_iota(jnp.int32, (SQ_SH, WIN), 1)
            mask = jnp.abs(rows - cols) <= 128
            ctx_parts = []
            for hh in range(HQ_SH):
                q_bh = xq[:, hh * DH:(hh + 1) * DH]
                k_bh = kb16[hh, pl.ds(s_b, WIN), :]
                v_bh = vb16[hh, pl.ds(s_b, WIN), :]
                s = lax.dot_general(
                    q_bh, k_bh, (((1,), (1,)), ((), ())),
                    preferred_element_type=F32,
                ) * SCALE
                s = jnp.where(mask, s, NEG)
                m = jnp.max(s, axis=1, keepdims=True)
                w = jnp.exp(s - m)
                w = (w / jnp.sum(w, axis=1, keepdims=True)).astype(BF)
                ctx_parts.append(
                    jnp.dot(w, v_bh, preferred_element_type=F32).astype(BF)
                )
            ctx = jnp.concatenate(ctx_parts, axis=1)
            dst[:, :] = jnp.dot(
                ctx, wob[:, :], preferred_element_type=F32
            ).astype(BF)


        d_ag_r.wait_recv()
        compute_chunk(agb[0, :, :], (my_pos + 3) % N_DEV, part.at[1])
        d_rs_l.start()

        d_ag_d.wait_recv()
        compute_chunk(agb[2, :, :], (my_pos + 2) % N_DEV, part.at[2])
        d_rs_d.start()

        d_ag_l.wait_recv()
        compute_chunk(agb[1, :, :], (my_pos + 1) % N_DEV, part.at[3])
        d_rs_r.start()

        compute_chunk(x_ref[:, :], my_pos, part.at[0])

        d_rs_r.wait_recv()
        d_rs_l.wait_recv()
        d_rs_d.wait_recv()
        out_ref[0, :, :] = (
            (part[0, :, :].astype(F32) + rsb[0, :, :].astype(F32))
            + (rsb[1, :, :].astype(F32) + rsb[2, :, :].astype(F32))
        )

        for d in [d_ag_r, d_ag_l, d_ag_d, d_rs_r, d_rs_l, d_rs_d]:
            d.wait_send()

    out = pl.pallas_call(
        body,
        out_shape=jax.ShapeDtypeStruct((1, SQ_SH, D_MODEL), F32),
        in_specs=[
            pl.BlockSpec(memory_space=pltpu.VMEM),
            pl.BlockSpec(memory_space=pl.ANY),
            pl.BlockSpec(memory_space=pl.ANY),
            pl.BlockSpec(memory_space=pl.ANY),
            pl.BlockSpec(memory_space=pl.ANY),
        ],
        out_specs=pl.BlockSpec(memory_space=pltpu.VMEM),
        scratch_shapes=[
            pltpu.VMEM((HQ_SH, KV_LEN, DH), F32),
            pltpu.VMEM((HQ_SH, KV_LEN, DH), F32),
            pltpu.VMEM((HQ_SH, KV_LEN, DH), BF),
            pltpu.VMEM((HQ_SH, KV_LEN, DH), BF),
            pltpu.VMEM((D_MODEL, D_MODEL), F32),
            pltpu.VMEM((D_MODEL, D_MODEL), BF),
            pltpu.VMEM((D_MODEL, D_MODEL), F32),
            pltpu.VMEM((D_MODEL, D_MODEL), BF),
            pltpu.VMEM((3, SQ_SH, D_MODEL), BF),
            pltpu.VMEM((3, SQ_SH, D_MODEL), BF),
            pltpu.VMEM((4, SQ_SH, D_MODEL), BF),
            pltpu.SemaphoreType.DMA((HQ_SH,)),
            pltpu.SemaphoreType.DMA((HQ_SH,)),
            pltpu.SemaphoreType.DMA((2,)),
            pltpu.SemaphoreType.DMA((3,)),
            pltpu.SemaphoreType.DMA((3,)),
            pltpu.SemaphoreType.DMA((3,)),
            pltpu.SemaphoreType.DMA((3,)),
        ],
        compiler_params=pltpu.CompilerParams(
            collective_id=0, vmem_limit_bytes=100 * 1024 * 1024
        ),
    )(x2, Wq, K_ext, V_ext, Wo)
    return out
